# baseline (speedup 1.0000x reference)
import functools

import jax
import jax.numpy as jnp
from jax import lax
from jax.experimental import pallas as pl
from jax.experimental.pallas import tpu as pltpu
from jax.experimental.pallas import tpu_sc as plsc

NC = 2
NS = 16
NW = NC * NS
CH = 128
G = 16


def _zero_fill(setter, ch, d):
  q = d // 16

  def st(i, carry):
    setter(i // q, 16 * lax.rem(i, q))
    return carry

  lax.fori_loop(0, ch * q, st, 0)


def _zero_shared(zbuf, shared, row0, rpt):
  ch = zbuf.shape[0]
  for k in range(rpt // ch):
    pltpu.sync_copy(zbuf, shared.at[pl.ds(row0 + k * ch, ch)])
  rem = rpt % ch
  if rem:
    pltpu.sync_copy(
        zbuf.at[pl.ds(0, rem)], shared.at[pl.ds(row0 + (rpt // ch) * ch, rem)])


def _sc_mesh():
  return plsc.VectorSubcoreMesh(
      core_axis_name="c", subcore_axis_name="s", num_cores=NC, num_subcores=NS
  )


def _make_deg_kernel(NP, E_pad, D):
  nch = E_pad // (NW * CH)
  rpt = NP // NS

  @functools.partial(
      pl.kernel,
      mesh=_sc_mesh(),
      out_type=jax.ShapeDtypeStruct((NC, NP, D), jnp.float32),
      scratch_types=[
          pltpu.VMEM((nch, CH), jnp.int32),
          pltpu.VMEM((CH, D), jnp.float32),
          pltpu.VMEM((CH, D), jnp.float32),
          pltpu.VMEM_SHARED((NP, D), jnp.float32),
      ],
  )
  def deg_kernel(col_hbm, ones_hbm, out_hbm, cidx, ones_v, zbuf, dacc):
    c = lax.axis_index("c")
    s = lax.axis_index("s")
    w = c * NS + s
    rows = pl.ds(s * rpt, rpt)
    def _setz(r, o):
      zbuf[r, pl.ds(o, 16)] = jnp.zeros((16,), jnp.float32)

    _zero_fill(_setz, CH, D)
    _zero_shared(zbuf, dacc, s * rpt, rpt)
    pltpu.sync_copy(ones_hbm, ones_v)
    pltpu.sync_copy(col_hbm.at[pl.ds(w * nch, nch)], cidx)
    plsc.subcore_barrier()

    def chunk(j, carry):
      pltpu.sync_copy(ones_v, dacc.at[cidx.at[j]], add=True)
      return carry

    lax.fori_loop(0, nch, chunk, 0)
    plsc.subcore_barrier()
    pltpu.sync_copy(dacc.at[rows], out_hbm.at[c, rows])

  return deg_kernel


def _make_scatter_kernel(NP, E_pad, D):
  nch = E_pad // (NW * CH)
  ngr = nch // G
  rpt = NP // NS

  @functools.partial(
      pl.kernel,
      mesh=_sc_mesh(),
      out_type=jax.ShapeDtypeStruct((NC, NP, D), jnp.float32),
      scratch_types=[
          pltpu.VMEM((2, G, CH), jnp.int32),
          pltpu.VMEM((2, G, CH), jnp.int32),
          pltpu.VMEM((2, CH, D), jnp.float32),
          pltpu.VMEM_SHARED((NP, D), jnp.float32),
          pltpu.SemaphoreType.DMA,
          pltpu.SemaphoreType.DMA,
          pltpu.SemaphoreType.DMA,
      ],
  )
  def scatter_kernel(z_hbm, row_hbm, col_hbm, out_hbm,
                     ridx, cidx, msg, acc, sem_g0, sem_g1, sem_i):
    c = lax.axis_index("c")
    s = lax.axis_index("s")
    w = c * NS + s
    base = w * nch
    rows = pl.ds(s * rpt, rpt)
    def _setz(r, o):
      msg[0, r, pl.ds(o, 16)] = jnp.zeros((16,), jnp.float32)

    _zero_fill(_setz, CH, D)
    _zero_shared(msg.at[0], acc, s * rpt, rpt)

    def start_idx(g, gb):
      pltpu.async_copy(row_hbm.at[pl.ds(base + g * G, G)], ridx.at[gb], sem_i)
      pltpu.async_copy(col_hbm.at[pl.ds(base + g * G, G)], cidx.at[gb], sem_i)

    def wait_idx(g, gb):
      pltpu.make_async_copy(
          row_hbm.at[pl.ds(base + g * G, G)], ridx.at[gb], sem_i).wait()
      pltpu.make_async_copy(
          col_hbm.at[pl.ds(base + g * G, G)], cidx.at[gb], sem_i).wait()

    start_idx(0, 0)

    def gather(gb, j, buf, sem):
      pltpu.async_copy(z_hbm.at[ridx.at[gb, j]], msg.at[buf], sem)

    def scatter(gb, j, buf):
      pltpu.sync_copy(msg.at[buf], acc.at[cidx.at[gb, j]], add=True)

    def group(g, carry):
      gb = lax.rem(g, 2)
      wait_idx(g, gb)

      @pl.when(g + 1 < ngr)
      def _():
        start_idx(g + 1, 1 - gb)

      gather(gb, 0, 0, sem_g0)

      def pair(t, carry2):
        pltpu.async_copy(z_hbm.at[ridx.at[gb, 2 * t + 1]], msg.at[1], sem_g1)
        pltpu.make_async_copy(
            z_hbm.at[ridx.at[gb, 2 * t]], msg.at[0], sem_g0).wait()
        scatter(gb, 2 * t, 0)

        @pl.when(2 * t + 2 < G)
        def _():
          gather(gb, 2 * t + 2, 0, sem_g0)

        pltpu.make_async_copy(
            z_hbm.at[ridx.at[gb, 2 * t + 1]], msg.at[1], sem_g1).wait()
        scatter(gb, 2 * t + 1, 1)
        return carry2

      lax.fori_loop(0, G // 2, pair, 0)
      return carry

    plsc.subcore_barrier()
    lax.fori_loop(0, ngr, group, 0)
    plsc.subcore_barrier()
    pltpu.sync_copy(acc.at[rows], out_hbm.at[c, rows])

  return scatter_kernel


def _dinv(d0_blk, d1_blk):
  d = d0_blk[:, 0:1] + d1_blk[:, 0:1]
  return jnp.where(d > 0.0, lax.rsqrt(jnp.where(d > 0.0, d, 1.0)), 0.0)


def _prep_body(d0, d1, x, z_out):
  z_out[...] = x[...] * _dinv(d0, d1)


def _combine_z_body(d0, d1, p0, p1, z_out):
  dinv = _dinv(d0, d1)
  z_out[...] = dinv * dinv * (p0[...] + p1[...])


def _total_body(scale, d0, d1, z, t_in, t_out):
  d = d0[:, 0:1] + d1[:, 0:1]
  t_out[...] = (t_in[...] + z[...] * jnp.sqrt(d)) * scale


def _tc_prep(d0, d1, x0, NP, D, B):
  return pl.pallas_call(
      _prep_body,
      grid=(NP // B,),
      in_specs=[
          pl.BlockSpec((B, 16), lambda i: (i, 0)),
          pl.BlockSpec((B, 16), lambda i: (i, 0)),
          pl.BlockSpec((B, D), lambda i: (i, 0)),
      ],
      out_specs=pl.BlockSpec((B, D), lambda i: (i, 0)),
      out_shape=jax.ShapeDtypeStruct((NP, D), jnp.float32),
  )(d0, d1, x0)


def _tc_combine_z(d0, d1, p0, p1, NP, D, B):
  spec = pl.BlockSpec((B, D), lambda i: (i, 0))
  spec16 = pl.BlockSpec((B, 16), lambda i: (i, 0))
  return pl.pallas_call(
      _combine_z_body,
      grid=(NP // B,),
      in_specs=[spec16, spec16, spec, spec],
      out_specs=spec,
      out_shape=jax.ShapeDtypeStruct((NP, D), jnp.float32),
  )(d0, d1, p0, p1)


def _tc_total(d0, d1, z, total, scale, NP, D, B):
  spec = pl.BlockSpec((B, D), lambda i: (i, 0))
  spec16 = pl.BlockSpec((B, 16), lambda i: (i, 0))
  return pl.pallas_call(
      functools.partial(_total_body, scale),
      grid=(NP // B,),
      in_specs=[spec16, spec16, spec, spec],
      out_specs=spec,
      out_shape=jax.ShapeDtypeStruct((NP, D), jnp.float32),
  )(d0, d1, z, total)


@jax.jit
def kernel(user_weight, item_weight, edge_index):
  NU, D = user_weight.shape
  NI = item_weight.shape[0]
  N = NU + NI
  E = edge_index.shape[1]
  NP = -(-(N + 1) // 128) * 128
  npad = NP - N
  assert NP % (8 * NS) == 0 and D % 16 == 0

  epw = -(-E // (NW * G * CH)) * G * CH
  E_pad = epw * NW
  pad_ids = (jnp.arange(E_pad - E, dtype=jnp.int32) % npad) + N
  row = jnp.concatenate([edge_index[0], pad_ids])
  col = jnp.concatenate([edge_index[1], pad_ids])
  row2 = row.reshape(E_pad // CH, CH)
  col2 = col.reshape(E_pad // CH, CH)

  x0 = jnp.concatenate(
      [user_weight, item_weight, jnp.zeros((npad, D), jnp.float32)], axis=0
  )
  onesD = jnp.ones((CH, D), jnp.float32)

  B = NP // 4 if (NP // 4) % 8 == 0 else NP
  num_layers = 4

  degP = _make_deg_kernel(NP, E_pad, D)(col2, onesD)
  d0, d1 = degP[0, :, :16], degP[1, :, :16]

  z = _tc_prep(d0, d1, x0, NP, D, B)
  total = x0
  sc_scatter = _make_scatter_kernel(NP, E_pad, D)
  for l in range(num_layers):
    P = sc_scatter(z, row2, col2)
    scale = 1.0 / ((num_layers + 1) ** 2) if l == num_layers - 1 else 1.0
    z = _tc_combine_z(d0, d1, P[0], P[1], NP, D, B)
    total = _tc_total(d0, d1, z, total, scale, NP, D, B)

  return total[:NU], total[NU:N]


# speedup vs baseline: 1.0248x; 1.0248x over previous
import functools

import jax
import jax.numpy as jnp
from jax import lax
from jax.experimental import pallas as pl
from jax.experimental.pallas import tpu as pltpu
from jax.experimental.pallas import tpu_sc as plsc

NC = 2
NS = 16
NW = NC * NS
CH = 128
G = 16


def _sc_mesh():
  return plsc.VectorSubcoreMesh(
      core_axis_name="c", subcore_axis_name="s", num_cores=NC, num_subcores=NS
  )


def _make_deg_kernel(NP, E_pad, D):
  nch = E_pad // (NW * CH)
  rpt = NP // NS

  @functools.partial(
      pl.kernel,
      mesh=_sc_mesh(),
      out_type=jax.ShapeDtypeStruct((NC, NP, D), jnp.float32),
      scratch_types=[
          pltpu.VMEM((nch, CH), jnp.int32),
          pltpu.VMEM((CH, D), jnp.float32),
          pltpu.VMEM_SHARED((NP, D), jnp.float32),
      ],
  )
  def deg_kernel(col_hbm, ones_hbm, zeros_hbm, out_hbm, cidx, ones_v, dacc):
    c = lax.axis_index("c")
    s = lax.axis_index("s")
    w = c * NS + s
    rows = pl.ds(s * rpt, rpt)
    pltpu.sync_copy(zeros_hbm.at[rows], dacc.at[rows])
    pltpu.sync_copy(ones_hbm, ones_v)
    pltpu.sync_copy(col_hbm.at[pl.ds(w * nch, nch)], cidx)
    plsc.subcore_barrier()

    def chunk(j, carry):
      pltpu.sync_copy(ones_v, dacc.at[cidx.at[j]], add=True)
      return carry

    lax.fori_loop(0, nch, chunk, 0)
    plsc.subcore_barrier()
    pltpu.sync_copy(dacc.at[rows], out_hbm.at[c, rows])

  return deg_kernel


def _make_scatter_kernel(NP, E_pad, D):
  nch = E_pad // (NW * CH)
  ngr = nch // G
  rpt = NP // NS

  @functools.partial(
      pl.kernel,
      mesh=_sc_mesh(),
      out_type=jax.ShapeDtypeStruct((NC, NP, D), jnp.float32),
      scratch_types=[
          pltpu.VMEM((2, G, CH), jnp.int32),
          pltpu.VMEM((2, G, CH), jnp.int32),
          pltpu.VMEM((2, CH, D), jnp.float32),
          pltpu.VMEM_SHARED((NP, D), jnp.float32),
          pltpu.SemaphoreType.DMA,
          pltpu.SemaphoreType.DMA,
          pltpu.SemaphoreType.DMA,
      ],
  )
  def scatter_kernel(z_hbm, row_hbm, col_hbm, zeros_hbm, out_hbm,
                     ridx, cidx, msg, acc, sem_g0, sem_g1, sem_i):
    c = lax.axis_index("c")
    s = lax.axis_index("s")
    w = c * NS + s
    base = w * nch
    rows = pl.ds(s * rpt, rpt)
    pltpu.sync_copy(zeros_hbm.at[rows], acc.at[rows])

    def start_idx(g, gb):
      pltpu.async_copy(row_hbm.at[pl.ds(base + g * G, G)], ridx.at[gb], sem_i)
      pltpu.async_copy(col_hbm.at[pl.ds(base + g * G, G)], cidx.at[gb], sem_i)

    def wait_idx(g, gb):
      pltpu.make_async_copy(
          row_hbm.at[pl.ds(base + g * G, G)], ridx.at[gb], sem_i).wait()
      pltpu.make_async_copy(
          col_hbm.at[pl.ds(base + g * G, G)], cidx.at[gb], sem_i).wait()

    start_idx(0, 0)

    def gather(gb, j, buf, sem):
      pltpu.async_copy(z_hbm.at[ridx.at[gb, j]], msg.at[buf], sem)

    def scatter(gb, j, buf):
      pltpu.sync_copy(msg.at[buf], acc.at[cidx.at[gb, j]], add=True)

    def group(g, carry):
      gb = lax.rem(g, 2)
      wait_idx(g, gb)

      @pl.when(g + 1 < ngr)
      def _():
        start_idx(g + 1, 1 - gb)

      gather(gb, 0, 0, sem_g0)

      def pair(t, carry2):
        pltpu.async_copy(z_hbm.at[ridx.at[gb, 2 * t + 1]], msg.at[1], sem_g1)
        pltpu.make_async_copy(
            z_hbm.at[ridx.at[gb, 2 * t]], msg.at[0], sem_g0).wait()
        scatter(gb, 2 * t, 0)

        @pl.when(2 * t + 2 < G)
        def _():
          gather(gb, 2 * t + 2, 0, sem_g0)

        pltpu.make_async_copy(
            z_hbm.at[ridx.at[gb, 2 * t + 1]], msg.at[1], sem_g1).wait()
        scatter(gb, 2 * t + 1, 1)
        return carry2

      lax.fori_loop(0, G // 2, pair, 0)
      return carry

    plsc.subcore_barrier()
    lax.fori_loop(0, ngr, group, 0)
    plsc.subcore_barrier()
    pltpu.sync_copy(acc.at[rows], out_hbm.at[c, rows])

  return scatter_kernel


def _dinv(d_blk):
  d = d_blk[:, 0:1]
  return jnp.where(d > 0.0, lax.rsqrt(jnp.where(d > 0.0, d, 1.0)), 0.0)


def _prep_body(d0f, d1f, x, z_out, d16_out):
  d16 = d0f[:, :16] + d1f[:, :16]
  d16_out[...] = d16
  dinv = jnp.where(d16[:, 0:1] > 0.0,
                   lax.rsqrt(jnp.where(d16[:, 0:1] > 0.0, d16[:, 0:1], 1.0)),
                   0.0)
  z_out[...] = x[...] * dinv


def _combine_body(scale, d16, p0, p1, t_in, t_out, z_out):
  dinv = _dinv(d16)
  x = dinv * (p0[...] + p1[...])
  t_out[...] = (t_in[...] + x) * scale
  z_out[...] = dinv * x


def _tc_prep(d0f, d1f, x0, NP, D, B):
  spec = pl.BlockSpec((B, D), lambda i: (i, 0))
  spec16 = pl.BlockSpec((B, 16), lambda i: (i, 0))
  return pl.pallas_call(
      _prep_body,
      grid=(NP // B,),
      in_specs=[spec, spec, spec],
      out_specs=[spec, spec16],
      out_shape=[
          jax.ShapeDtypeStruct((NP, D), jnp.float32),
          jax.ShapeDtypeStruct((NP, 16), jnp.float32),
      ],
  )(d0f, d1f, x0)


def _tc_combine(d16, p0, p1, total, scale, NP, D, B):
  spec = pl.BlockSpec((B, D), lambda i: (i, 0))
  spec16 = pl.BlockSpec((B, 16), lambda i: (i, 0))
  return pl.pallas_call(
      functools.partial(_combine_body, scale),
      grid=(NP // B,),
      in_specs=[spec16, spec, spec, spec],
      out_specs=[spec, spec],
      out_shape=[
          jax.ShapeDtypeStruct((NP, D), jnp.float32),
          jax.ShapeDtypeStruct((NP, D), jnp.float32),
      ],
  )(d16, p0, p1, total)


@jax.jit
def kernel(user_weight, item_weight, edge_index):
  NU, D = user_weight.shape
  NI = item_weight.shape[0]
  N = NU + NI
  E = edge_index.shape[1]
  NP = -(-(N + 1) // 128) * 128
  npad = NP - N
  assert NP % (8 * NS) == 0 and D % 16 == 0

  epw = -(-E // (NW * G * CH)) * G * CH
  E_pad = epw * NW
  pad_ids = (jnp.arange(E_pad - E, dtype=jnp.int32) % npad) + N
  row = jnp.concatenate([edge_index[0], pad_ids])
  col = jnp.concatenate([edge_index[1], pad_ids])
  row2 = row.reshape(E_pad // CH, CH)
  col2 = col.reshape(E_pad // CH, CH)

  x0 = jnp.concatenate(
      [user_weight, item_weight, jnp.zeros((npad, D), jnp.float32)], axis=0
  )
  zerosD = jnp.zeros((NP, D), jnp.float32)
  onesD = jnp.ones((CH, D), jnp.float32)

  B = NP // 4 if (NP // 4) % 8 == 0 else NP
  num_layers = 4

  degP = _make_deg_kernel(NP, E_pad, D)(col2, onesD, zerosD)

  z, d16 = _tc_prep(degP[0], degP[1], x0, NP, D, B)
  total = x0
  sc_scatter = _make_scatter_kernel(NP, E_pad, D)
  for l in range(num_layers):
    P = sc_scatter(z, row2, col2, zerosD)
    scale = 1.0 / ((num_layers + 1) ** 2) if l == num_layers - 1 else 1.0
    total, z = _tc_combine(d16, P[0], P[1], total, scale, NP, D, B)

  return total[:NU], total[NU:N]


# speedup vs baseline: 1.0811x; 1.0550x over previous
import functools

import jax
import jax.numpy as jnp
from jax import lax
from jax.experimental import pallas as pl
from jax.experimental.pallas import tpu as pltpu
from jax.experimental.pallas import tpu_sc as plsc

NC = 2
NS = 16
NW = NC * NS
CH = 128
G = 16


def _sc_mesh():
  return plsc.VectorSubcoreMesh(
      core_axis_name="c", subcore_axis_name="s", num_cores=NC, num_subcores=NS
  )


def _make_deg_kernel(NP, E_pad, D):
  nch = E_pad // (NW * CH)
  rpt = NP // NS

  @functools.partial(
      pl.kernel,
      mesh=_sc_mesh(),
      out_type=jax.ShapeDtypeStruct((NC, NP, D), jnp.float32),
      scratch_types=[
          pltpu.VMEM((nch, CH), jnp.int32),
          pltpu.VMEM((CH, D), jnp.float32),
          pltpu.VMEM_SHARED((NP, D), jnp.float32),
      ],
  )
  def deg_kernel(col_hbm, ones_hbm, zeros_hbm, out_hbm, cidx, ones_v, dacc):
    c = lax.axis_index("c")
    s = lax.axis_index("s")
    w = c * NS + s
    rows = pl.ds(s * rpt, rpt)
    pltpu.sync_copy(zeros_hbm.at[rows], dacc.at[rows])
    pltpu.sync_copy(ones_hbm, ones_v)
    pltpu.sync_copy(col_hbm.at[pl.ds(w * nch, nch)], cidx)
    plsc.subcore_barrier()

    def chunk(j, carry):
      pltpu.sync_copy(ones_v, dacc.at[cidx.at[j]], add=True)
      return carry

    lax.fori_loop(0, nch, chunk, 0)
    plsc.subcore_barrier()
    pltpu.sync_copy(dacc.at[rows], out_hbm.at[c, rows])

  return deg_kernel


def _make_scatter_kernel(NP, E_pad, D):
  nch = E_pad // (NW * CH)
  ngr = nch // G
  rpt = NP // NS

  @functools.partial(
      pl.kernel,
      mesh=_sc_mesh(),
      out_type=jax.ShapeDtypeStruct((NC, NP, D), jnp.float32),
      scratch_types=[
          pltpu.VMEM((2, G, CH), jnp.int32),
          pltpu.VMEM((2, G, CH), jnp.int32),
          pltpu.VMEM((2, CH, D), jnp.float32),
          pltpu.VMEM_SHARED((NP, D), jnp.float32),
          pltpu.SemaphoreType.DMA,
          pltpu.SemaphoreType.DMA,
          pltpu.SemaphoreType.DMA,
      ],
  )
  def scatter_kernel(z_hbm, row_hbm, col_hbm, zeros_hbm, out_hbm,
                     ridx, cidx, msg, acc, sem_g0, sem_g1, sem_i):
    c = lax.axis_index("c")
    s = lax.axis_index("s")
    w = c * NS + s
    base = w * nch
    rows = pl.ds(s * rpt, rpt)
    pltpu.sync_copy(zeros_hbm.at[rows], acc.at[rows])

    def start_idx(g, gb):
      pltpu.async_copy(row_hbm.at[pl.ds(base + g * G, G)], ridx.at[gb], sem_i)
      pltpu.async_copy(col_hbm.at[pl.ds(base + g * G, G)], cidx.at[gb], sem_i)

    def wait_idx(g, gb):
      pltpu.make_async_copy(
          row_hbm.at[pl.ds(base + g * G, G)], ridx.at[gb], sem_i).wait()
      pltpu.make_async_copy(
          col_hbm.at[pl.ds(base + g * G, G)], cidx.at[gb], sem_i).wait()

    start_idx(0, 0)

    def gather(gb, j, buf, sem):
      pltpu.async_copy(z_hbm.at[ridx.at[gb, j]], msg.at[buf], sem)

    def scatter(gb, j, buf):
      pltpu.sync_copy(msg.at[buf], acc.at[cidx.at[gb, j]], add=True)

    def group(g, carry):
      gb = lax.rem(g, 2)
      wait_idx(g, gb)

      @pl.when(g + 1 < ngr)
      def _():
        start_idx(g + 1, 1 - gb)

      gather(gb, 0, 0, sem_g0)

      def pair(t, carry2):
        pltpu.async_copy(z_hbm.at[ridx.at[gb, 2 * t + 1]], msg.at[1], sem_g1)
        pltpu.make_async_copy(
            z_hbm.at[ridx.at[gb, 2 * t]], msg.at[0], sem_g0).wait()
        scatter(gb, 2 * t, 0)

        @pl.when(2 * t + 2 < G)
        def _():
          gather(gb, 2 * t + 2, 0, sem_g0)

        pltpu.make_async_copy(
            z_hbm.at[ridx.at[gb, 2 * t + 1]], msg.at[1], sem_g1).wait()
        scatter(gb, 2 * t + 1, 1)
        return carry2

      lax.fori_loop(0, G // 2, pair, 0)
      return carry

    plsc.subcore_barrier()
    lax.fori_loop(0, ngr, group, 0)
    plsc.subcore_barrier()
    pltpu.sync_copy(acc.at[rows], out_hbm.at[c, rows])

  return scatter_kernel


def _dinv(d_blk):
  d = d_blk[:, 0:1]
  return jnp.where(d > 0.0, lax.rsqrt(jnp.where(d > 0.0, d, 1.0)), 0.0)


def _prep_body(dP, x, z_out, d16_out):
  d16 = dP[0, :, :16] + dP[1, :, :16]
  d16_out[...] = d16
  dinv = jnp.where(d16[:, 0:1] > 0.0,
                   lax.rsqrt(jnp.where(d16[:, 0:1] > 0.0, d16[:, 0:1], 1.0)),
                   0.0)
  z_out[...] = x[...] * dinv


def _combine_body(scale, last, d16, P, t_in, t_out, *z_out):
  dinv = _dinv(d16)
  x = dinv * (P[0] + P[1])
  t_out[...] = (t_in[...] + x) * scale
  if not last:
    z_out[0][...] = dinv * x


def _tc_prep(degP, x0, NP, D, B):
  spec = pl.BlockSpec((B, D), lambda i: (i, 0))
  spec16 = pl.BlockSpec((B, 16), lambda i: (i, 0))
  specP = pl.BlockSpec((2, B, D), lambda i: (0, i, 0))
  return pl.pallas_call(
      _prep_body,
      grid=(NP // B,),
      in_specs=[specP, spec],
      out_specs=[spec, spec16],
      out_shape=[
          jax.ShapeDtypeStruct((NP, D), jnp.float32),
          jax.ShapeDtypeStruct((NP, 16), jnp.float32),
      ],
  )(degP, x0)


def _tc_combine(d16, P, total, scale, last, NP, D, B):
  spec = pl.BlockSpec((B, D), lambda i: (i, 0))
  spec16 = pl.BlockSpec((B, 16), lambda i: (i, 0))
  specP = pl.BlockSpec((2, B, D), lambda i: (0, i, 0))
  out_specs = [spec] if last else [spec, spec]
  out_shape = [jax.ShapeDtypeStruct((NP, D), jnp.float32)] * len(out_specs)
  res = pl.pallas_call(
      functools.partial(_combine_body, scale, last),
      grid=(NP // B,),
      in_specs=[spec16, specP, spec],
      out_specs=out_specs,
      out_shape=out_shape,
  )(d16, P, total)
  return res if not last else (res[0], None)


@jax.jit
def kernel(user_weight, item_weight, edge_index):
  NU, D = user_weight.shape
  NI = item_weight.shape[0]
  N = NU + NI
  E = edge_index.shape[1]
  NP = -(-(N + 1) // 128) * 128
  npad = NP - N
  assert NP % (8 * NS) == 0 and D % 16 == 0

  epw = -(-E // (NW * G * CH)) * G * CH
  E_pad = epw * NW
  pad_ids = (jnp.arange(E_pad - E, dtype=jnp.int32) % npad) + N
  row = jnp.concatenate([edge_index[0], pad_ids])
  col = jnp.concatenate([edge_index[1], pad_ids])
  row2 = row.reshape(E_pad // CH, CH)
  col2 = col.reshape(E_pad // CH, CH)

  x0 = jnp.concatenate(
      [user_weight, item_weight, jnp.zeros((npad, D), jnp.float32)], axis=0
  )
  zerosD = jnp.zeros((NP, D), jnp.float32)
  onesD = jnp.ones((CH, D), jnp.float32)

  B = NP // 4 if (NP // 4) % 8 == 0 else NP
  num_layers = 4

  degP = _make_deg_kernel(NP, E_pad, D)(col2, onesD, zerosD)

  z, d16 = _tc_prep(degP, x0, NP, D, B)
  total = x0
  sc_scatter = _make_scatter_kernel(NP, E_pad, D)
  for l in range(num_layers):
    P = sc_scatter(z, row2, col2, zerosD)
    last = l == num_layers - 1
    scale = 1.0 / ((num_layers + 1) ** 2) if last else 1.0
    total, z = _tc_combine(d16, P, total, scale, last, NP, D, B)

  return total[:NU], total[NU:N]


# speedup vs baseline: 1.1694x; 1.0816x over previous
import functools

import jax
import jax.numpy as jnp
from jax import lax
from jax.experimental import pallas as pl
from jax.experimental.pallas import tpu as pltpu
from jax.experimental.pallas import tpu_sc as plsc

NC = 2
NS = 16
NW = NC * NS
CH = 128
G = 16


def _sc_mesh():
  return plsc.VectorSubcoreMesh(
      core_axis_name="c", subcore_axis_name="s", num_cores=NC, num_subcores=NS
  )


def _make_deg_kernel(NP, E_pad):
  nch = E_pad // (NW * CH)
  rpt = NP // NS

  @functools.partial(
      pl.kernel,
      mesh=_sc_mesh(),
      out_type=jax.ShapeDtypeStruct((NC * NP,), jnp.float32),
      scratch_types=[
          pltpu.VMEM((nch, CH), jnp.int32),
          pltpu.VMEM((CH,), jnp.float32),
          pltpu.VMEM((rpt,), jnp.float32),
          pltpu.VMEM_SHARED((NP,), jnp.float32),
      ],
  )
  def deg_kernel(col_hbm, ones_hbm, zeros_hbm, out_hbm, cidx, ones_v, zbuf,
                 dacc):
    c = lax.axis_index("c")
    s = lax.axis_index("s")
    w = c * NS + s
    rows = pl.ds(s * rpt, rpt)
    pltpu.sync_copy(zeros_hbm.at[rows], zbuf)
    pltpu.sync_copy(zbuf, dacc.at[rows])
    pltpu.sync_copy(ones_hbm, ones_v)
    pltpu.sync_copy(col_hbm.at[pl.ds(w * nch, nch)], cidx)
    plsc.subcore_barrier()

    def chunk(j, carry):
      pltpu.sync_copy(ones_v, dacc.at[cidx.at[j]], add=True)
      return carry

    lax.fori_loop(0, nch, chunk, 0)
    plsc.subcore_barrier()
    pltpu.sync_copy(dacc.at[rows], zbuf)
    pltpu.sync_copy(zbuf, out_hbm.at[pl.ds(c * NP + s * rpt, rpt)])

  return deg_kernel


def _make_scatter_kernel(NP, E_pad, D):
  nch = E_pad // (NW * CH)
  ngr = nch // G
  rpt = NP // NS

  @functools.partial(
      pl.kernel,
      mesh=_sc_mesh(),
      out_type=jax.ShapeDtypeStruct((NC, NP, D), jnp.float32),
      scratch_types=[
          pltpu.VMEM((2, G, CH), jnp.int32),
          pltpu.VMEM((2, G, CH), jnp.int32),
          pltpu.VMEM((2, CH, D), jnp.float32),
          pltpu.VMEM_SHARED((NP, D), jnp.float32),
          pltpu.SemaphoreType.DMA,
          pltpu.SemaphoreType.DMA,
          pltpu.SemaphoreType.DMA,
      ],
  )
  def scatter_kernel(z_hbm, row_hbm, col_hbm, zeros_hbm, out_hbm,
                     ridx, cidx, msg, acc, sem_g0, sem_g1, sem_i):
    c = lax.axis_index("c")
    s = lax.axis_index("s")
    w = c * NS + s
    base = w * nch
    rows = pl.ds(s * rpt, rpt)
    pltpu.sync_copy(zeros_hbm.at[rows], acc.at[rows])

    def start_idx(g, gb):
      pltpu.async_copy(row_hbm.at[pl.ds(base + g * G, G)], ridx.at[gb], sem_i)
      pltpu.async_copy(col_hbm.at[pl.ds(base + g * G, G)], cidx.at[gb], sem_i)

    def wait_idx(g, gb):
      pltpu.make_async_copy(
          row_hbm.at[pl.ds(base + g * G, G)], ridx.at[gb], sem_i).wait()
      pltpu.make_async_copy(
          col_hbm.at[pl.ds(base + g * G, G)], cidx.at[gb], sem_i).wait()

    start_idx(0, 0)

    def gather(gb, j, buf, sem):
      pltpu.async_copy(z_hbm.at[ridx.at[gb, j]], msg.at[buf], sem)

    def scatter(gb, j, buf):
      pltpu.sync_copy(msg.at[buf], acc.at[cidx.at[gb, j]], add=True)

    def group(g, carry):
      gb = lax.rem(g, 2)
      wait_idx(g, gb)

      @pl.when(g + 1 < ngr)
      def _():
        start_idx(g + 1, 1 - gb)

      gather(gb, 0, 0, sem_g0)

      def pair(t, carry2):
        pltpu.async_copy(z_hbm.at[ridx.at[gb, 2 * t + 1]], msg.at[1], sem_g1)
        pltpu.make_async_copy(
            z_hbm.at[ridx.at[gb, 2 * t]], msg.at[0], sem_g0).wait()
        scatter(gb, 2 * t, 0)

        @pl.when(2 * t + 2 < G)
        def _():
          gather(gb, 2 * t + 2, 0, sem_g0)

        pltpu.make_async_copy(
            z_hbm.at[ridx.at[gb, 2 * t + 1]], msg.at[1], sem_g1).wait()
        scatter(gb, 2 * t + 1, 1)
        return carry2

      lax.fori_loop(0, G // 2, pair, 0)
      return carry

    plsc.subcore_barrier()
    lax.fori_loop(0, ngr, group, 0)
    plsc.subcore_barrier()
    pltpu.sync_copy(acc.at[rows], out_hbm.at[c, rows])

  return scatter_kernel


def _prep_body(d0, d1, x, z_out, dv_out):
  d = d0[...] + d1[...]
  dinv = jnp.where(d > 0.0, lax.rsqrt(jnp.where(d > 0.0, d, 1.0)), 0.0)
  dv_out[...] = dinv
  z_out[...] = x[...] * dinv


def _combine_body(scale, last, dv, P, t_in, t_out, *z_out):
  dinv = dv[...]
  x = dinv * (P[0] + P[1])
  t_out[...] = (t_in[...] + x) * scale
  if not last:
    z_out[0][...] = dinv * x


def _tc_prep(d0c, d1c, x0, NP, D, B):
  spec = pl.BlockSpec((B, D), lambda i: (i, 0))
  spec1 = pl.BlockSpec((B, 1), lambda i: (i, 0))
  return pl.pallas_call(
      _prep_body,
      grid=(NP // B,),
      in_specs=[spec1, spec1, spec],
      out_specs=[spec, spec1],
      out_shape=[
          jax.ShapeDtypeStruct((NP, D), jnp.float32),
          jax.ShapeDtypeStruct((NP, 1), jnp.float32),
      ],
  )(d0c, d1c, x0)


def _tc_combine(dv, P, total, scale, last, NP, D, B):
  spec = pl.BlockSpec((B, D), lambda i: (i, 0))
  spec1 = pl.BlockSpec((B, 1), lambda i: (i, 0))
  specP = pl.BlockSpec((2, B, D), lambda i: (0, i, 0))
  out_specs = [spec] if last else [spec, spec]
  out_shape = [jax.ShapeDtypeStruct((NP, D), jnp.float32)] * len(out_specs)
  res = pl.pallas_call(
      functools.partial(_combine_body, scale, last),
      grid=(NP // B,),
      in_specs=[spec1, specP, spec],
      out_specs=out_specs,
      out_shape=out_shape,
  )(dv, P, total)
  return res if not last else (res[0], None)


@jax.jit
def kernel(user_weight, item_weight, edge_index):
  NU, D = user_weight.shape
  NI = item_weight.shape[0]
  N = NU + NI
  E = edge_index.shape[1]
  NP = -(-(N + 1) // 128) * 128
  npad = NP - N
  assert NP % (8 * NS) == 0 and D % 16 == 0

  epw = -(-E // (NW * G * CH)) * G * CH
  E_pad = epw * NW
  pad_ids = (jnp.arange(E_pad - E, dtype=jnp.int32) % npad) + N
  row = jnp.concatenate([edge_index[0], pad_ids])
  col = jnp.concatenate([edge_index[1], pad_ids])
  row2 = row.reshape(E_pad // CH, CH)
  col2 = col.reshape(E_pad // CH, CH)

  x0 = jnp.concatenate(
      [user_weight, item_weight, jnp.zeros((npad, D), jnp.float32)], axis=0
  )
  zerosD = jnp.zeros((NP, D), jnp.float32)
  ones1 = jnp.ones((CH,), jnp.float32)
  zeros1 = jnp.zeros((NP,), jnp.float32)

  B = NP // 4 if (NP // 4) % 8 == 0 else NP
  num_layers = 4

  degF = _make_deg_kernel(NP, E_pad)(col2, ones1, zeros1)
  d0c = degF[:NP].reshape(NP, 1)
  d1c = degF[NP:].reshape(NP, 1)

  z, dv = _tc_prep(d0c, d1c, x0, NP, D, B)
  total = x0
  sc_scatter = _make_scatter_kernel(NP, E_pad, D)
  for l in range(num_layers):
    P = sc_scatter(z, row2, col2, zerosD)
    last = l == num_layers - 1
    scale = 1.0 / ((num_layers + 1) ** 2) if last else 1.0
    total, z = _tc_combine(dv, P, total, scale, last, NP, D, B)

  return total[:NU], total[NU:N]
